# Initial kernel scaffold; baseline (speedup 1.0000x reference)
import jax
import jax.numpy as jnp
from jax.experimental import pallas as pl


def kernel(inputs, embedding):
    raise NotImplementedError("write your pallas kernel here")


# speedup vs baseline: 1.5768x; 1.5768x over previous
import functools

import jax
import jax.numpy as jnp
from jax import lax
from jax.experimental import pallas as pl
from jax.experimental.pallas import tpu as pltpu
from jax.experimental.pallas import tpu_sc as plsc

NC = 2
NS = 16
NW = NC * NS


def _gather_kernel(n_rows, d, chunk):
    per_w = n_rows // NW
    n_ch = per_w // chunk
    mesh = plsc.VectorSubcoreMesh(
        core_axis_name="c", subcore_axis_name="s",
        num_cores=NC, num_subcores=NS)

    @functools.partial(
        pl.kernel,
        out_type=jax.ShapeDtypeStruct((n_rows, d), jnp.float32),
        mesh=mesh,
        compiler_params=pltpu.CompilerParams(use_tc_tiling_on_sc=False),
        scratch_types=[
            pltpu.VMEM((chunk,), jnp.int32),
            pltpu.VMEM((chunk,), jnp.int32),
            pltpu.VMEM((chunk, d), jnp.float32),
            pltpu.VMEM((chunk, d), jnp.float32),
            pltpu.SemaphoreType.DMA((2,)),
            pltpu.SemaphoreType.DMA((2,)),
            pltpu.SemaphoreType.DMA((2,)),
        ],
    )
    def body(idx_hbm, tab_hbm, out_hbm,
             idx_v0, idx_v1, rows_v0, rows_v1, sem_i, sem_g, sem_o):
        idx_v = (idx_v0, idx_v1)
        rows_v = (rows_v0, rows_v1)
        wid = lax.axis_index("s") * NC + lax.axis_index("c")
        base = wid * per_w

        def idx_copy(g, s):
            return pltpu.async_copy(
                idx_hbm.at[pl.ds(base + g * chunk, chunk)], idx_v[s],
                sem_i.at[s])

        def gather_copy(s):
            return pltpu.async_copy(
                tab_hbm.at[idx_v[s]], rows_v[s], sem_g.at[s])

        def store_copy(g, s):
            return pltpu.async_copy(
                rows_v[s], out_hbm.at[pl.ds(base + g * chunk, chunk)],
                sem_o.at[s])

        idx_cp = [idx_copy(0, 0)]
        if n_ch > 1:
            idx_cp.append(idx_copy(1, 1))
        idx_cp[0].wait()
        gathers = [gather_copy(0), None]
        stores = [None, None]
        for g in range(n_ch):
            s = g & 1
            o = 1 - s
            if g + 1 < n_ch:
                if stores[o] is not None:
                    stores[o].wait()
                idx_cp[o].wait()
                gathers[o] = gather_copy(o)
            gathers[s].wait()
            stores[s] = store_copy(g, s)
            if g + 2 < n_ch:
                idx_cp[s] = idx_copy(g + 2, s)
        for st in stores:
            if st is not None:
                st.wait()

    return body


def kernel(inputs, embedding):
    b, f = inputs.shape
    v, d = embedding.shape
    idx = inputs.reshape(-1).astype(jnp.int32)
    n_rows = b * f
    out = _gather_kernel(n_rows, d, 1664)(idx, embedding)
    return out.reshape(b, f, d)
